# baseline (speedup 1.0000x reference)
import functools

import jax
import jax.numpy as jnp
from jax import lax
from jax.experimental import pallas as pl
from jax.experimental.pallas import tpu as pltpu
from jax.experimental.pallas import tpu_sc as plsc

B = 2048
K = 1024
D = 64
BLK_B = 256


def _argmin_body(z_ref, cb_ref, idx_ref, idxd_ref, cbp_ref):
    z = z_ref[...]
    cb = cb_ref[...]
    cbp_ref[:, :D] = cb
    cbp_ref[:, D:] = cb
    cbt = cb.T
    cbn = jnp.sum(cbt * cbt, axis=0)
    dots2 = jnp.dot(z, -2.0 * cbt, preferred_element_type=jnp.float32,
                    precision=lax.Precision.HIGHEST)
    scores = cbn[None, :] + dots2
    m = jnp.min(scores, axis=1, keepdims=True)
    ids = lax.broadcasted_iota(jnp.int32, scores.shape, 1)
    idx = jnp.min(jnp.where(scores == m, ids, K), axis=1)
    idx_ref[...] = idx.reshape(idx_ref.shape)
    idxd_ref[...] = (idx * 2).reshape(idxd_ref.shape)


def _nearest_indices(z_e, codebook):
    return pl.pallas_call(
        _argmin_body,
        out_shape=(
            jax.ShapeDtypeStruct((16, 128), jnp.int32),
            jax.ShapeDtypeStruct((16, 128), jnp.int32),
            jax.ShapeDtypeStruct((K, 2 * D), jnp.float32),
        ),
    )(z_e, codebook)


@functools.lru_cache(maxsize=1)
def _make_sc_gather():
    info = plsc.get_sparse_core_info()
    nc, ns = info.num_cores, info.num_subcores
    nw = nc * ns
    b_per_w = B // nw
    mesh = plsc.VectorSubcoreMesh(core_axis_name="c", subcore_axis_name="s")

    @functools.partial(
        pl.kernel,
        mesh=mesh,
        compiler_params=pltpu.CompilerParams(use_tc_tiling_on_sc=False),
        out_type=jax.ShapeDtypeStruct((B, D), jnp.float32),
        scratch_types=[
            pltpu.VMEM((b_per_w,), jnp.int32),
            pltpu.VMEM((b_per_w, D), jnp.float32),
            pltpu.SemaphoreType.DMA,
        ],
    )
    def gather_k(table_hbm, idx_hbm, out_hbm, idx_v, rows_v, sem):
        wid = lax.axis_index("s") * nc + lax.axis_index("c")
        base = wid * b_per_w
        pltpu.sync_copy(idx_hbm.at[base // 128, pl.ds((base % 128), b_per_w)],
                        idx_v)
        pltpu.async_copy(table_hbm.at[idx_v], rows_v, sem).wait()
        pltpu.sync_copy(rows_v, out_hbm.at[pl.ds(base, b_per_w)])

    return gather_k


def kernel(z_e, codebook):
    idx2, idxd, cbp = _nearest_indices(z_e, codebook)
    z_q = _make_sc_gather()(cbp.reshape(2 * K, D), idxd)
    return (z_q, idx2.reshape(B))


# speedup vs baseline: 1.0158x; 1.0158x over previous
import functools

import jax
import jax.numpy as jnp
from jax import lax
from jax.experimental import pallas as pl
from jax.experimental.pallas import tpu as pltpu
from jax.experimental.pallas import tpu_sc as plsc

B = 2048
K = 1024
D = 64
BLK_B = 256


def _argmin_body(z_ref, cb_ref, idx_ref):
    z = z_ref[...]
    cbt = cb_ref[...].T
    cbn = jnp.sum(cbt * cbt, axis=0)
    dots2 = jnp.dot(z, -2.0 * cbt, preferred_element_type=jnp.float32,
                    precision=lax.Precision.HIGHEST)
    scores = cbn[None, :] + dots2
    m = jnp.min(scores, axis=1, keepdims=True)
    ids = lax.broadcasted_iota(jnp.int32, scores.shape, 1)
    idx = jnp.min(jnp.where(scores == m, ids, K), axis=1)
    idx_ref[...] = idx.reshape(idx_ref.shape)


def _nearest_indices(z_e, codebook):
    return pl.pallas_call(
        _argmin_body,
        grid=(2,),
        in_specs=[
            pl.BlockSpec((B // 2, D), lambda i: (i, 0)),
            pl.BlockSpec((K, D), lambda i: (0, 0)),
        ],
        out_specs=pl.BlockSpec((8, 128), lambda i: (i, 0)),
        out_shape=jax.ShapeDtypeStruct((16, 128), jnp.int32),
    )(z_e, codebook)


@functools.lru_cache(maxsize=1)
def _make_sc_gather():
    info = plsc.get_sparse_core_info()
    nc, ns = info.num_cores, info.num_subcores
    nw = nc * ns
    b_per_w = B // nw
    mesh = plsc.VectorSubcoreMesh(core_axis_name="c", subcore_axis_name="s")

    @functools.partial(
        pl.kernel,
        mesh=mesh,
        compiler_params=pltpu.CompilerParams(use_tc_tiling_on_sc=False),
        out_type=jax.ShapeDtypeStruct((B, D), jnp.float32),
        scratch_types=[
            pltpu.VMEM((b_per_w,), jnp.int32),
            pltpu.VMEM((b_per_w, D), jnp.float32),
            pltpu.SemaphoreType.DMA,
        ],
    )
    def gather_k(table_hbm, idx_hbm, out_hbm, idx_v, rows_v, sem):
        wid = lax.axis_index("s") * nc + lax.axis_index("c")
        base = wid * b_per_w
        pltpu.sync_copy(idx_hbm.at[base // 128, pl.ds((base % 128), b_per_w)],
                        idx_v)
        pltpu.async_copy(table_hbm.at[idx_v], rows_v, sem).wait()
        pltpu.sync_copy(rows_v, out_hbm.at[pl.ds(base, b_per_w)])

    return gather_k


def kernel(z_e, codebook):
    idx2 = _nearest_indices(z_e, codebook)
    z_q = _make_sc_gather()(codebook, idx2)
    return (z_q, idx2.reshape(B))
